# Initial kernel scaffold; baseline (speedup 1.0000x reference)
import jax
import jax.numpy as jnp
from jax.experimental import pallas as pl


def kernel(x, ln_gamma, ln_beta, W1, b1, W2, b2):
    raise NotImplementedError("write your pallas kernel here")


# speedup vs baseline: 1.3404x; 1.3404x over previous
import functools

import jax
import jax.numpy as jnp
from jax.experimental import pallas as pl
from jax.experimental.pallas import tpu as pltpu

TOPK = 8


def _router_body(x_ref, g_ref, bt_ref, w1_ref, b1_ref, w2_ref, b2_ref,
                 ti_ref, tv_ref, aux_ref,
                 xn_ref, acc_ref, ps_ref,
                 *, n_chunks, n_tblocks, n_tokens, n_experts):
    t = pl.program_id(0)
    n = pl.program_id(1)

    @pl.when(n == 0)
    def _prologue():
        xx = x_ref[...]
        mu = jnp.mean(xx, axis=1, keepdims=True)
        xc = xx - mu
        var = jnp.mean(xc * xc, axis=1, keepdims=True)
        xn = xc / jnp.sqrt(var + 1e-5) * g_ref[...] + bt_ref[...]
        xn_ref[...] = xn.astype(jnp.bfloat16)
        acc_ref[...] = jnp.zeros_like(acc_ref)

    @pl.when((t == 0) & (n == 0))
    def _init_psum():
        ps_ref[...] = jnp.zeros_like(ps_ref)

    h = jnp.dot(xn_ref[...], w1_ref[...],
                preferred_element_type=jnp.float32)
    h = jnp.maximum(h + b1_ref[...], 0.0).astype(jnp.bfloat16)
    acc_ref[...] += jnp.dot(h, w2_ref[...],
                            preferred_element_type=jnp.float32)

    @pl.when(n == n_chunks - 1)
    def _epilogue():
        logits = acc_ref[...] + b2_ref[...]
        m = jnp.max(logits, axis=1, keepdims=True)
        e = jnp.exp(logits - m)
        p = e / jnp.sum(e, axis=1, keepdims=True)
        ps_ref[...] += jnp.sum(p, axis=0, keepdims=True)

        iota = jax.lax.broadcasted_iota(jnp.int32, p.shape, 1)
        v = p
        tvs, tis = [], []
        for _ in range(TOPK):
            mk = jnp.max(v, axis=1, keepdims=True)
            ik = jnp.min(jnp.where(v == mk, iota, n_experts), axis=1,
                         keepdims=True)
            tvs.append(mk)
            tis.append(ik)
            v = jnp.where(iota == ik, -1.0, v)
        tv = jnp.concatenate(tvs, axis=1)
        ti_ref[...] = jnp.concatenate(tis, axis=1)
        tv_ref[...] = tv / jnp.sum(tv, axis=1, keepdims=True)

        @pl.when(t == n_tblocks - 1)
        def _finalize_aux():
            s = ps_ref[...] / n_tokens
            aux = jnp.sum(s * jnp.log(s * n_experts + 1e-9), axis=1,
                          keepdims=True)
            aux_ref[...] = aux


def kernel(x, ln_gamma, ln_beta, W1, b1, W2, b2):
    B, S, H = x.shape
    E = W2.shape[1]
    BS = B * S

    TB = 512 if BS % 512 == 0 else BS
    NC = 512 if H % 512 == 0 else H
    n_tblocks = BS // TB
    n_chunks = H // NC

    x2 = x.reshape(BS, H)
    g2 = ln_gamma.reshape(1, H)
    bt2 = ln_beta.reshape(1, H)
    b1_2 = b1.reshape(1, H)
    b2_2 = b2.reshape(1, E)

    body = functools.partial(_router_body, n_chunks=n_chunks,
                             n_tblocks=n_tblocks, n_tokens=BS, n_experts=E)

    ti, tv, aux = pl.pallas_call(
        body,
        grid=(n_tblocks, n_chunks),
        in_specs=[
            pl.BlockSpec((TB, H), lambda t, n: (t, 0)),
            pl.BlockSpec((1, H), lambda t, n: (0, 0)),
            pl.BlockSpec((1, H), lambda t, n: (0, 0)),
            pl.BlockSpec((H, NC), lambda t, n: (0, n)),
            pl.BlockSpec((1, NC), lambda t, n: (0, n)),
            pl.BlockSpec((NC, E), lambda t, n: (n, 0)),
            pl.BlockSpec((1, E), lambda t, n: (0, 0)),
        ],
        out_specs=[
            pl.BlockSpec((TB, TOPK), lambda t, n: (t, 0)),
            pl.BlockSpec((TB, TOPK), lambda t, n: (t, 0)),
            pl.BlockSpec((1, 1), lambda t, n: (0, 0)),
        ],
        out_shape=[
            jax.ShapeDtypeStruct((BS, TOPK), jnp.int32),
            jax.ShapeDtypeStruct((BS, TOPK), jnp.float32),
            jax.ShapeDtypeStruct((1, 1), jnp.float32),
        ],
        scratch_shapes=[
            pltpu.VMEM((TB, H), jnp.bfloat16),
            pltpu.VMEM((TB, E), jnp.float32),
            pltpu.VMEM((1, E), jnp.float32),
        ],
    )(x2, g2, bt2, W1.astype(jnp.bfloat16), b1_2, W2.astype(jnp.bfloat16),
      b2_2)

    return (ti.reshape(B, S, TOPK), tv.reshape(B, S, TOPK),
            aux.reshape(()))


# speedup vs baseline: 1.3428x; 1.0018x over previous
import functools

import jax
import jax.numpy as jnp
from jax.experimental import pallas as pl
from jax.experimental.pallas import tpu as pltpu

TOPK = 8


def _router_body(x_ref, g_ref, bt_ref, w1_ref, b1_ref, w2_ref, b2_ref,
                 ti_ref, tv_ref, ps_ref,
                 xn_ref, acc_ref,
                 *, n_chunks, n_experts):
    n = pl.program_id(1)

    @pl.when(n == 0)
    def _prologue():
        xx = x_ref[...]
        mu = jnp.mean(xx, axis=1, keepdims=True)
        xc = xx - mu
        var = jnp.mean(xc * xc, axis=1, keepdims=True)
        xn = xc / jnp.sqrt(var + 1e-5) * g_ref[...] + bt_ref[...]
        xn_ref[...] = xn.astype(jnp.bfloat16)
        acc_ref[...] = jnp.zeros_like(acc_ref)

    h = jnp.dot(xn_ref[...], w1_ref[...],
                preferred_element_type=jnp.float32)
    h = jnp.maximum(h + b1_ref[...], 0.0).astype(jnp.bfloat16)
    acc_ref[...] += jnp.dot(h, w2_ref[...],
                            preferred_element_type=jnp.float32)

    @pl.when(n == n_chunks - 1)
    def _epilogue():
        logits = acc_ref[...] + b2_ref[...]
        m = jnp.max(logits, axis=1, keepdims=True)
        e = jnp.exp(logits - m)
        p = e / jnp.sum(e, axis=1, keepdims=True)
        ps_ref[...] = jnp.sum(p, axis=0, keepdims=True)[None]

        iota = jax.lax.broadcasted_iota(jnp.int32, p.shape, 1)
        v = p
        tvs, tis = [], []
        for _ in range(TOPK):
            mk = jnp.max(v, axis=1, keepdims=True)
            ik = jnp.min(jnp.where(v == mk, iota, n_experts), axis=1,
                         keepdims=True)
            tvs.append(mk)
            tis.append(ik)
            v = jnp.where(iota == ik, -1.0, v)
        tv = jnp.concatenate(tvs, axis=1)
        ti_ref[...] = jnp.concatenate(tis, axis=1)
        tv_ref[...] = tv / jnp.sum(tv, axis=1, keepdims=True)


def _aux_body(ps_ref, aux_ref, *, n_tokens, n_experts):
    s = jnp.sum(ps_ref[...], axis=(0, 1)).reshape(1, -1) / n_tokens
    aux_ref[...] = jnp.sum(s * jnp.log(s * n_experts + 1e-9), axis=1,
                           keepdims=True)


def kernel(x, ln_gamma, ln_beta, W1, b1, W2, b2):
    B, S, H = x.shape
    E = W2.shape[1]
    BS = B * S

    TB = 512 if BS % 512 == 0 else BS
    NC = 512 if H % 512 == 0 else H
    n_tblocks = BS // TB
    n_chunks = H // NC

    x2 = x.reshape(BS, H)
    g2 = ln_gamma.reshape(1, H)
    bt2 = ln_beta.reshape(1, H)
    b1_2 = b1.reshape(1, H)
    b2_2 = b2.reshape(1, E)

    body = functools.partial(_router_body, n_chunks=n_chunks, n_experts=E)

    ti, tv, ps = pl.pallas_call(
        body,
        grid=(n_tblocks, n_chunks),
        in_specs=[
            pl.BlockSpec((TB, H), lambda t, n: (t, 0)),
            pl.BlockSpec((1, H), lambda t, n: (0, 0)),
            pl.BlockSpec((1, H), lambda t, n: (0, 0)),
            pl.BlockSpec((H, NC), lambda t, n: (0, n)),
            pl.BlockSpec((1, NC), lambda t, n: (0, n)),
            pl.BlockSpec((NC, E), lambda t, n: (n, 0)),
            pl.BlockSpec((1, E), lambda t, n: (0, 0)),
        ],
        out_specs=[
            pl.BlockSpec((TB, TOPK), lambda t, n: (t, 0)),
            pl.BlockSpec((TB, TOPK), lambda t, n: (t, 0)),
            pl.BlockSpec((1, 1, E), lambda t, n: (t, 0, 0)),
        ],
        out_shape=[
            jax.ShapeDtypeStruct((BS, TOPK), jnp.int32),
            jax.ShapeDtypeStruct((BS, TOPK), jnp.float32),
            jax.ShapeDtypeStruct((n_tblocks, 1, E), jnp.float32),
        ],
        scratch_shapes=[
            pltpu.VMEM((TB, H), jnp.bfloat16),
            pltpu.VMEM((TB, E), jnp.float32),
        ],
        compiler_params=pltpu.CompilerParams(
            dimension_semantics=("parallel", "arbitrary")),
    )(x2, g2, bt2, W1.astype(jnp.bfloat16), b1_2, W2.astype(jnp.bfloat16),
      b2_2)

    aux = pl.pallas_call(
        functools.partial(_aux_body, n_tokens=BS, n_experts=E),
        out_shape=jax.ShapeDtypeStruct((1, 1), jnp.float32),
    )(ps)

    return (ti.reshape(B, S, TOPK), tv.reshape(B, S, TOPK),
            aux.reshape(()))
